# Initial kernel scaffold; baseline (speedup 1.0000x reference)
import functools

import jax
import jax.numpy as jnp
from jax import lax
from jax.experimental import pallas as pl
from jax.experimental.pallas import tpu as pltpu
from jax.experimental.pallas import tpu_sc as plsc

N = 10000
E = 160000
D_IN = 256
D_HID = 512
D_OUT = 256

NC = 2
NS = 16
EPT = E // NS
NB = EPT // 128
TAIL = EPT - NB * 128
SLAB_STEP = 624
SLAB = 640

_f32 = jnp.float32
_i32 = jnp.int32


def _rsqrt16(d):
    i = plsc.bitcast(d, _i32)
    i = 0x5F3759DF - lax.shift_right_logical(i, 1)
    y = plsc.bitcast(i, _f32)
    for _ in range(3):
        y = y * (1.5 - 0.5 * d * y * y)
    return y


def _make_deg_kernel():
    mesh = plsc.VectorSubcoreMesh(core_axis_name="c", subcore_axis_name="s")

    @functools.partial(
        pl.kernel,
        out_type=jax.ShapeDtypeStruct((N,), _f32),
        mesh=mesh,
        scratch_types=[
            pltpu.VMEM((128,), _i32),
            pltpu.VMEM((16,), _i32),
            pltpu.VMEM((128,), _f32),
            pltpu.VMEM((SLAB,), _f32),
            pltpu.VMEM_SHARED((N,), _f32),
        ],
    )
    def deg_kernel(col_hbm, zeros_hbm, dis_hbm, idx_v, idx_t, ones_v, buf_v, acc):
        c = lax.axis_index("c")
        s = lax.axis_index("s")
        slab0 = s * SLAB_STEP

        @pl.when(c == 0)
        def _zero():
            pltpu.sync_copy(zeros_hbm, acc.at[pl.ds(slab0, SLAB)])

        plsc.subcore_barrier()

        @pl.when(c == 0)
        def _scatter():
            for k in range(8):
                ones_v[pl.ds(k * 16, 16)] = jnp.full((16,), 1.0, _f32)
            base = s * EPT

            def step(i, carry):
                off = base + i * 128
                pltpu.sync_copy(col_hbm.at[pl.ds(off, 128)], idx_v)
                pltpu.sync_copy(ones_v, acc.at[idx_v], add=True)
                return carry

            lax.fori_loop(0, NB, step, 0)
            pltpu.sync_copy(col_hbm.at[pl.ds(base + NB * 128, TAIL)], idx_t)
            pltpu.sync_copy(ones_v.at[pl.ds(0, TAIL)], acc.at[idx_t], add=True)

        plsc.subcore_barrier()

        @pl.when(c == 0)
        def _dis():
            pltpu.sync_copy(acc.at[pl.ds(slab0, SLAB)], buf_v)
            for j in range(SLAB // 16):
                d = buf_v[pl.ds(j * 16, 16)] + 1.0
                buf_v[pl.ds(j * 16, 16)] = _rsqrt16(d)
            pltpu.sync_copy(buf_v, dis_hbm.at[pl.ds(slab0, SLAB)])

    return deg_kernel


_deg_kernel = _make_deg_kernel()


def _make_scatter_kernel(nchunk):
    cpc = nchunk // NC
    mesh = plsc.VectorSubcoreMesh(core_axis_name="c", subcore_axis_name="s")

    @functools.partial(
        pl.kernel,
        out_type=jax.ShapeDtypeStruct((nchunk * N, 128), _f32),
        mesh=mesh,
        scratch_types=[
            pltpu.VMEM((128,), _i32),
            pltpu.VMEM((128,), _i32),
            pltpu.VMEM((16,), _i32),
            pltpu.VMEM((16,), _i32),
            pltpu.VMEM((128, 128), _f32),
            pltpu.VMEM_SHARED((N, 128), _f32),
        ],
    )
    def scatter_kernel(g_hbm, row_hbm, col_hbm, zeros_hbm, agg_hbm,
                       idx_s, idx_d, idx_st, idx_dt, rows_v, acc):
        c = lax.axis_index("c")
        s = lax.axis_index("s")
        slab0 = s * SLAB_STEP
        base = s * EPT

        for j in range(cpc):
            chunk = c * cpc + j
            goff = chunk * N
            pltpu.sync_copy(zeros_hbm, acc.at[pl.ds(slab0, SLAB)])
            plsc.subcore_barrier()

            def step(i, carry):
                off = base + i * 128
                pltpu.sync_copy(row_hbm.at[pl.ds(off, 128)], idx_s)
                pltpu.sync_copy(col_hbm.at[pl.ds(off, 128)], idx_d)
                for k in range(8):
                    idx_s[pl.ds(k * 16, 16)] = idx_s[pl.ds(k * 16, 16)] + goff
                pltpu.sync_copy(g_hbm.at[idx_s], rows_v)
                pltpu.sync_copy(rows_v, acc.at[idx_d], add=True)
                return carry

            lax.fori_loop(0, NB, step, 0)

            toff = base + NB * 128
            pltpu.sync_copy(row_hbm.at[pl.ds(toff, TAIL)], idx_st)
            pltpu.sync_copy(col_hbm.at[pl.ds(toff, TAIL)], idx_dt)
            idx_st[...] = idx_st[...] + goff
            pltpu.sync_copy(g_hbm.at[idx_st], rows_v.at[pl.ds(0, TAIL)])
            pltpu.sync_copy(rows_v.at[pl.ds(0, TAIL)], acc.at[idx_dt], add=True)

            plsc.subcore_barrier()
            pltpu.sync_copy(acc.at[pl.ds(slab0, SLAB)],
                            agg_hbm.at[pl.ds(goff + slab0, SLAB)])
            plsc.subcore_barrier()

    return scatter_kernel


_scatter4 = _make_scatter_kernel(4)
_scatter2 = _make_scatter_kernel(2)


def _tc1_body(x_ref, w_ref, dis_ref, out_ref):
    h = jnp.dot(x_ref[...], w_ref[...], preferred_element_type=_f32)
    out_ref[0] = h * dis_ref[...]


def _tc1(x, W1, dis_col):
    return pl.pallas_call(
        _tc1_body,
        grid=(N // 1000, D_HID // 128),
        in_specs=[
            pl.BlockSpec((1000, D_IN), lambda r, c: (r, 0)),
            pl.BlockSpec((D_IN, 128), lambda r, c: (0, c)),
            pl.BlockSpec((1000, 1), lambda r, c: (r, 0)),
        ],
        out_specs=pl.BlockSpec((1, 1000, 128), lambda r, c: (c, r, 0)),
        out_shape=jax.ShapeDtypeStruct((D_HID // 128, N, 128), _f32),
    )(x, W1, dis_col)


def _tc2_body(agg_ref, g_ref, dis_ref, b1_ref, w2_ref, out_ref):
    dis = dis_ref[...]
    z = jnp.concatenate(
        [jax.nn.relu(dis * (agg_ref[k] + g_ref[k]) + b1_ref[k]) for k in range(4)],
        axis=1)
    h2 = jnp.dot(z, w2_ref[...], preferred_element_type=_f32)
    out_ref[0] = h2 * dis


def _tc2(agg1, g1, dis_col, b1c, W2):
    return pl.pallas_call(
        _tc2_body,
        grid=(N // 1000, D_OUT // 128),
        in_specs=[
            pl.BlockSpec((4, 1000, 128), lambda r, c: (0, r, 0)),
            pl.BlockSpec((4, 1000, 128), lambda r, c: (0, r, 0)),
            pl.BlockSpec((1000, 1), lambda r, c: (r, 0)),
            pl.BlockSpec((4, 128), lambda r, c: (0, 0)),
            pl.BlockSpec((D_HID, 128), lambda r, c: (0, c)),
        ],
        out_specs=pl.BlockSpec((1, 1000, 128), lambda r, c: (c, r, 0)),
        out_shape=jax.ShapeDtypeStruct((D_OUT // 128, N, 128), _f32),
    )(agg1, g1, dis_col, b1c, W2)


def _tc3_body(agg_ref, g_ref, dis_ref, b2_ref, out_ref):
    r = pl.program_id(0)
    dis = dis_ref[...]
    y = jnp.concatenate(
        [dis * (agg_ref[k] + g_ref[k]) + b2_ref[k] for k in range(2)], axis=1)
    p = jnp.sum(y, axis=0, keepdims=True) * (1.0 / (N // 2))

    @pl.when(r == 0)
    def _():
        out_ref[...] = jnp.zeros((2, D_OUT), _f32)

    gid = r // 5
    mask = lax.broadcasted_iota(_i32, (2, D_OUT), 0) == gid
    out_ref[...] += jnp.where(mask, jnp.broadcast_to(p, (2, D_OUT)), 0.0)


def _tc3(agg2, g2, dis_col, b2c):
    return pl.pallas_call(
        _tc3_body,
        grid=(N // 1000,),
        in_specs=[
            pl.BlockSpec((2, 1000, 128), lambda r: (0, r, 0)),
            pl.BlockSpec((2, 1000, 128), lambda r: (0, r, 0)),
            pl.BlockSpec((1000, 1), lambda r: (r, 0)),
            pl.BlockSpec((2, 128), lambda r: (0, 0)),
        ],
        out_specs=pl.BlockSpec((2, D_OUT), lambda r: (0, 0)),
        out_shape=jax.ShapeDtypeStruct((2, D_OUT), _f32),
    )(agg2, g2, dis_col, b2c)


def kernel(x, edge_index, W1, b1, W2, b2):
    row = edge_index[0]
    col = edge_index[1]
    zeros1 = jnp.zeros((SLAB,), _f32)
    zeros2 = jnp.zeros((SLAB, 128), _f32)

    dis = _deg_kernel(col, zeros1)
    dis_col = dis.reshape(N, 1)

    g1 = _tc1(x, W1, dis_col)
    agg1 = _scatter4(g1.reshape(4 * N, 128), row, col, zeros2)
    g2 = _tc2(agg1.reshape(4, N, 128), g1, dis_col,
              b1.reshape(4, 128), W2)
    agg2 = _scatter2(g2.reshape(2 * N, 128), row, col, zeros2)
    return _tc3(agg2.reshape(2, N, 128), g2, dis_col, b2.reshape(2, 128))


# speedup vs baseline: 8.4090x; 8.4090x over previous
import functools

import jax
import jax.numpy as jnp
from jax import lax
from jax.experimental import pallas as pl
from jax.experimental.pallas import tpu as pltpu
from jax.experimental.pallas import tpu_sc as plsc

N = 10000
E = 160000
D_IN = 256
D_HID = 512
D_OUT = 256

NC = 2
NS = 16
EPT = E // NS
NB = EPT // 128
TAIL = EPT - NB * 128
SLAB_STEP = 624
SLAB = 640

_f32 = jnp.float32
_i32 = jnp.int32


def _rsqrt16(d):
    i = lax.bitcast_convert_type(d, _i32)
    i = 0x5F3759DF - lax.shift_right_logical(i, 1)
    y = lax.bitcast_convert_type(i, _f32)
    for _ in range(3):
        y = y * (1.5 - 0.5 * d * y * y)
    return y


@functools.lru_cache(maxsize=None)
def _make_deg_kernel():
    mesh = plsc.VectorSubcoreMesh(core_axis_name="c", subcore_axis_name="s")

    @functools.partial(
        pl.kernel,
        out_type=jax.ShapeDtypeStruct((N,), _f32),
        mesh=mesh,
        scratch_types=[
            pltpu.VMEM((128,), _i32),
            pltpu.VMEM((16,), _i32),
            pltpu.VMEM((128,), _f32),
            pltpu.VMEM((SLAB,), _f32),
            pltpu.VMEM_SHARED((N,), _f32),
        ],
    )
    def deg_kernel(col_hbm, dis_hbm, idx_v, idx_t, ones_v, buf_v, acc):
        c = lax.axis_index("c")
        s = lax.axis_index("s")
        slab0 = s * SLAB_STEP

        @pl.when(c == 0)
        def _zero():
            for j in range(SLAB // 16):
                buf_v[pl.ds(j * 16, 16)] = jnp.zeros((16,), _f32)
            pltpu.sync_copy(buf_v, acc.at[pl.ds(slab0, SLAB)])

        plsc.subcore_barrier()

        @pl.when(c == 0)
        def _scatter():
            for k in range(8):
                ones_v[pl.ds(k * 16, 16)] = jnp.full((16,), 1.0, _f32)
            base = s * EPT

            def step(i, carry):
                off = base + i * 128
                pltpu.sync_copy(col_hbm.at[pl.ds(off, 128)], idx_v)
                pltpu.sync_copy(ones_v, acc.at[idx_v], add=True)
                return carry

            lax.fori_loop(0, NB, step, 0)
            pltpu.sync_copy(col_hbm.at[pl.ds(base + NB * 128, TAIL)], idx_t)
            pltpu.sync_copy(ones_v.at[pl.ds(0, TAIL)], acc.at[idx_t], add=True)

        plsc.subcore_barrier()

        @pl.when(c == 0)
        def _dis():
            pltpu.sync_copy(acc.at[pl.ds(slab0, SLAB)], buf_v)
            for j in range(SLAB // 16):
                d = buf_v[pl.ds(j * 16, 16)] + 1.0
                buf_v[pl.ds(j * 16, 16)] = _rsqrt16(d)
            pltpu.sync_copy(buf_v, dis_hbm.at[pl.ds(slab0, SLAB)])

    return deg_kernel


@functools.lru_cache(maxsize=None)
def _make_scatter_kernel(nchunk):
    cpc = nchunk // NC
    mesh = plsc.VectorSubcoreMesh(core_axis_name="c", subcore_axis_name="s")

    @functools.partial(
        pl.kernel,
        out_type=jax.ShapeDtypeStruct((nchunk * N, 128), _f32),
        mesh=mesh,
        scratch_types=[
            pltpu.VMEM((128,), _i32),
            pltpu.VMEM((128,), _i32),
            pltpu.VMEM((16,), _i32),
            pltpu.VMEM((16,), _i32),
            pltpu.VMEM((128, 128), _f32),
            pltpu.VMEM_SHARED((N, 128), _f32),
        ],
    )
    def scatter_kernel(g_hbm, row_hbm, col_hbm, agg_hbm,
                       idx_s, idx_d, idx_st, idx_dt, rows_v, acc):
        c = lax.axis_index("c")
        s = lax.axis_index("s")
        slab0 = s * SLAB_STEP
        base = s * EPT

        def zfill(r, carry):
            for k in range(8):
                rows_v[r, pl.ds(k * 16, 16)] = jnp.zeros((16,), _f32)
            return carry

        for j in range(cpc):
            chunk = c * cpc + j
            goff = chunk * N
            lax.fori_loop(0, 128, zfill, 0)
            for p in range(SLAB // 128):
                pltpu.sync_copy(rows_v, acc.at[pl.ds(slab0 + p * 128, 128)])
            plsc.subcore_barrier()

            def step(i, carry):
                off = base + i * 128
                pltpu.sync_copy(row_hbm.at[pl.ds(off, 128)], idx_s)
                pltpu.sync_copy(col_hbm.at[pl.ds(off, 128)], idx_d)
                for k in range(8):
                    idx_s[pl.ds(k * 16, 16)] = idx_s[pl.ds(k * 16, 16)] + goff
                pltpu.sync_copy(g_hbm.at[idx_s], rows_v)
                pltpu.sync_copy(rows_v, acc.at[idx_d], add=True)
                return carry

            lax.fori_loop(0, NB, step, 0)

            toff = base + NB * 128
            pltpu.sync_copy(row_hbm.at[pl.ds(toff, TAIL)], idx_st)
            pltpu.sync_copy(col_hbm.at[pl.ds(toff, TAIL)], idx_dt)
            idx_st[...] = idx_st[...] + goff
            pltpu.sync_copy(g_hbm.at[idx_st], rows_v.at[pl.ds(0, TAIL)])
            pltpu.sync_copy(rows_v.at[pl.ds(0, TAIL)], acc.at[idx_dt], add=True)

            plsc.subcore_barrier()
            for p in range(SLAB // 128):
                pltpu.sync_copy(acc.at[pl.ds(slab0 + p * 128, 128)], rows_v)
                pltpu.sync_copy(rows_v,
                                agg_hbm.at[pl.ds(goff + slab0 + p * 128, 128)])
            plsc.subcore_barrier()

    return scatter_kernel


def _tc1_body(x_ref, w_ref, dis_ref, out_ref):
    h = jnp.dot(x_ref[...], w_ref[...], preferred_element_type=_f32)
    out_ref[0] = h * dis_ref[...]


def _tc1(x, W1, dis_col):
    return pl.pallas_call(
        _tc1_body,
        grid=(N // 1000, D_HID // 128),
        in_specs=[
            pl.BlockSpec((1000, D_IN), lambda r, c: (r, 0)),
            pl.BlockSpec((D_IN, 128), lambda r, c: (0, c)),
            pl.BlockSpec((1000, 1), lambda r, c: (r, 0)),
        ],
        out_specs=pl.BlockSpec((1, 1000, 128), lambda r, c: (c, r, 0)),
        out_shape=jax.ShapeDtypeStruct((D_HID // 128, N, 128), _f32),
    )(x, W1, dis_col)


def _tc2_body(agg_ref, g_ref, dis_ref, b1_ref, w2_ref, out_ref):
    dis = dis_ref[...]
    z = jnp.concatenate(
        [jax.nn.relu(dis * (agg_ref[k] + g_ref[k]) + b1_ref[k]) for k in range(4)],
        axis=1)
    h2 = jnp.dot(z, w2_ref[...], preferred_element_type=_f32)
    out_ref[0] = h2 * dis


def _tc2(agg1, g1, dis_col, b1c, W2):
    return pl.pallas_call(
        _tc2_body,
        grid=(N // 1000, D_OUT // 128),
        in_specs=[
            pl.BlockSpec((4, 1000, 128), lambda r, c: (0, r, 0)),
            pl.BlockSpec((4, 1000, 128), lambda r, c: (0, r, 0)),
            pl.BlockSpec((1000, 1), lambda r, c: (r, 0)),
            pl.BlockSpec((4, 128), lambda r, c: (0, 0)),
            pl.BlockSpec((D_HID, 128), lambda r, c: (0, c)),
        ],
        out_specs=pl.BlockSpec((1, 1000, 128), lambda r, c: (c, r, 0)),
        out_shape=jax.ShapeDtypeStruct((D_OUT // 128, N, 128), _f32),
    )(agg1, g1, dis_col, b1c, W2)


def _tc3_body(agg_ref, g_ref, dis_ref, b2_ref, out_ref):
    r = pl.program_id(0)
    dis = dis_ref[...]
    y = jnp.concatenate(
        [dis * (agg_ref[k] + g_ref[k]) + b2_ref[k] for k in range(2)], axis=1)
    p = jnp.sum(y, axis=0, keepdims=True) * (1.0 / (N // 2))

    @pl.when(r == 0)
    def _():
        out_ref[...] = jnp.zeros((2, D_OUT), _f32)

    gid = r // 5
    mask = lax.broadcasted_iota(_i32, (2, D_OUT), 0) == gid
    out_ref[...] += jnp.where(mask, jnp.broadcast_to(p, (2, D_OUT)), 0.0)


def _tc3(agg2, g2, dis_col, b2c):
    return pl.pallas_call(
        _tc3_body,
        grid=(N // 1000,),
        in_specs=[
            pl.BlockSpec((2, 1000, 128), lambda r: (0, r, 0)),
            pl.BlockSpec((2, 1000, 128), lambda r: (0, r, 0)),
            pl.BlockSpec((1000, 1), lambda r: (r, 0)),
            pl.BlockSpec((2, 128), lambda r: (0, 0)),
        ],
        out_specs=pl.BlockSpec((2, D_OUT), lambda r: (0, 0)),
        out_shape=jax.ShapeDtypeStruct((2, D_OUT), _f32),
    )(agg2, g2, dis_col, b2c)


def kernel(x, edge_index, W1, b1, W2, b2):
    row = edge_index[0]
    col = edge_index[1]

    dis = _make_deg_kernel()(col)
    dis_col = dis.reshape(N, 1)

    g1 = _tc1(x, W1, dis_col)
    agg1 = _make_scatter_kernel(4)(g1.reshape(4 * N, 128), row, col)
    g2 = _tc2(agg1.reshape(4, N, 128), g1, dis_col,
              b1.reshape(4, 128), W2)
    agg2 = _make_scatter_kernel(2)(g2.reshape(2 * N, 128), row, col)
    return _tc3(agg2.reshape(2, N, 128), g2, dis_col, b2.reshape(2, 128))


# speedup vs baseline: 12.4599x; 1.4817x over previous
import functools

import jax
import jax.numpy as jnp
from jax import lax
from jax.experimental import pallas as pl
from jax.experimental.pallas import tpu as pltpu
from jax.experimental.pallas import tpu_sc as plsc

N = 10000
E = 160000
D_IN = 256
D_HID = 512
D_OUT = 256

NC = 2
NS = 16
EPT = E // NS
NB = EPT // 128
TAIL = EPT - NB * 128
SLAB_STEP = 624
SLAB = 640
NBUF = 2

_f32 = jnp.float32
_i32 = jnp.int32


def _rsqrt16(d):
    i = lax.bitcast_convert_type(d, _i32)
    i = 0x5F3759DF - lax.shift_right_logical(i, 1)
    y = lax.bitcast_convert_type(i, _f32)
    for _ in range(3):
        y = y * (1.5 - 0.5 * d * y * y)
    return y


@functools.lru_cache(maxsize=None)
def _make_deg_kernel():
    mesh = plsc.VectorSubcoreMesh(core_axis_name="c", subcore_axis_name="s")

    @functools.partial(
        pl.kernel,
        out_type=jax.ShapeDtypeStruct((N,), _f32),
        mesh=mesh,
        scratch_types=[
            pltpu.VMEM((128,), _i32),
            pltpu.VMEM((16,), _i32),
            pltpu.VMEM((128,), _f32),
            pltpu.VMEM((SLAB,), _f32),
            pltpu.VMEM_SHARED((N,), _f32),
        ],
    )
    def deg_kernel(col_hbm, dis_hbm, idx_v, idx_t, ones_v, buf_v, acc):
        c = lax.axis_index("c")
        s = lax.axis_index("s")
        slab0 = s * SLAB_STEP

        @pl.when(c == 0)
        def _zero():
            for j in range(SLAB // 16):
                buf_v[pl.ds(j * 16, 16)] = jnp.zeros((16,), _f32)
            pltpu.sync_copy(buf_v, acc.at[pl.ds(slab0, SLAB)])

        plsc.subcore_barrier()

        @pl.when(c == 0)
        def _scatter():
            for k in range(8):
                ones_v[pl.ds(k * 16, 16)] = jnp.full((16,), 1.0, _f32)
            base = s * EPT

            def step(i, carry):
                off = base + i * 128
                pltpu.sync_copy(col_hbm.at[pl.ds(off, 128)], idx_v)
                pltpu.sync_copy(ones_v, acc.at[idx_v], add=True)
                return carry

            lax.fori_loop(0, NB, step, 0)
            pltpu.sync_copy(col_hbm.at[pl.ds(base + NB * 128, TAIL)], idx_t)
            pltpu.sync_copy(ones_v.at[pl.ds(0, TAIL)], acc.at[idx_t], add=True)

        plsc.subcore_barrier()

        @pl.when(c == 0)
        def _dis():
            pltpu.sync_copy(acc.at[pl.ds(slab0, SLAB)], buf_v)
            for j in range(SLAB // 16):
                d = buf_v[pl.ds(j * 16, 16)] + 1.0
                buf_v[pl.ds(j * 16, 16)] = _rsqrt16(d)
            pltpu.sync_copy(buf_v, dis_hbm.at[pl.ds(slab0, SLAB)])

    return deg_kernel


@functools.lru_cache(maxsize=None)
def _make_scatter_kernel(nchunk):
    cpc = nchunk // NC
    mesh = plsc.VectorSubcoreMesh(core_axis_name="c", subcore_axis_name="s")

    @functools.partial(
        pl.kernel,
        out_type=jax.ShapeDtypeStruct((nchunk * N, 128), _f32),
        mesh=mesh,
        scratch_types=[
            pltpu.VMEM((EPT,), _i32),
            pltpu.VMEM((NBUF, 128), _i32),
            pltpu.VMEM((16,), _i32),
            pltpu.VMEM((16,), _i32),
            pltpu.VMEM((NBUF, 128, 128), _f32),
            pltpu.VMEM_SHARED((N, 128), _f32),
            pltpu.SemaphoreType.DMA((NBUF,)),
        ],
    )
    def scatter_kernel(g_hbm, row_hbm, col_hbm, agg_hbm,
                       idx_s, idx_d, idx_st, idx_dt, bufs, acc, sems):
        c = lax.axis_index("c")
        s = lax.axis_index("s")
        slab0 = s * SLAB_STEP
        base = s * EPT

        pltpu.sync_copy(row_hbm.at[pl.ds(base, EPT)], idx_s)
        pltpu.sync_copy(row_hbm.at[pl.ds(base + NB * 128, TAIL)], idx_st)
        pltpu.sync_copy(col_hbm.at[pl.ds(base + NB * 128, TAIL)], idx_dt)

        def zfill(r, carry):
            for k in range(8):
                bufs[0, r, pl.ds(k * 16, 16)] = jnp.zeros((16,), _f32)
            return carry

        for j in range(cpc):
            chunk = c * cpc + j
            goff = chunk * N
            gview = g_hbm.at[pl.ds(goff, N)]

            lax.fori_loop(0, 128, zfill, 0)
            for p in range(SLAB // 128):
                pltpu.sync_copy(bufs.at[0], acc.at[pl.ds(slab0 + p * 128, 128)])
            plsc.subcore_barrier()

            def body(t, carry):
                i0 = t * NBUF
                hs = []
                for b in range(NBUF):
                    off = i0 + b
                    hs.append((
                        pltpu.async_copy(
                            col_hbm.at[pl.ds(base + off * 128, 128)],
                            idx_d.at[b], sems.at[b]),
                        pltpu.async_copy(
                            gview.at[idx_s.at[pl.ds(off * 128, 128)]],
                            bufs.at[b], sems.at[b]),
                    ))
                for b in range(NBUF):
                    hs[b][0].wait()
                    hs[b][1].wait()
                    pltpu.sync_copy(bufs.at[b], acc.at[idx_d.at[b]], add=True)
                return carry

            lax.fori_loop(0, NB // NBUF, body, 0)

            pltpu.sync_copy(gview.at[idx_st], bufs.at[0].at[pl.ds(0, TAIL)])
            pltpu.sync_copy(bufs.at[0].at[pl.ds(0, TAIL)],
                            acc.at[idx_dt], add=True)

            plsc.subcore_barrier()
            for p in range(SLAB // 128):
                pltpu.sync_copy(acc.at[pl.ds(slab0 + p * 128, 128)], bufs.at[0])
                pltpu.sync_copy(bufs.at[0],
                                agg_hbm.at[pl.ds(goff + slab0 + p * 128, 128)])
            plsc.subcore_barrier()

    return scatter_kernel


def _tc1_body(x_ref, w_ref, dis_ref, out_ref):
    h = jnp.dot(x_ref[...], w_ref[...], preferred_element_type=_f32)
    out_ref[0] = h * dis_ref[...]


def _tc1(x, W1, dis_col):
    return pl.pallas_call(
        _tc1_body,
        grid=(N // 1000, D_HID // 128),
        in_specs=[
            pl.BlockSpec((1000, D_IN), lambda r, c: (r, 0)),
            pl.BlockSpec((D_IN, 128), lambda r, c: (0, c)),
            pl.BlockSpec((1000, 1), lambda r, c: (r, 0)),
        ],
        out_specs=pl.BlockSpec((1, 1000, 128), lambda r, c: (c, r, 0)),
        out_shape=jax.ShapeDtypeStruct((D_HID // 128, N, 128), _f32),
    )(x, W1, dis_col)


def _tc2_body(agg_ref, g_ref, dis_ref, b1_ref, w2_ref, out_ref):
    dis = dis_ref[...]
    z = jnp.concatenate(
        [jax.nn.relu(dis * (agg_ref[k] + g_ref[k]) + b1_ref[k]) for k in range(4)],
        axis=1)
    h2 = jnp.dot(z, w2_ref[...], preferred_element_type=_f32)
    out_ref[0] = h2 * dis


def _tc2(agg1, g1, dis_col, b1c, W2):
    return pl.pallas_call(
        _tc2_body,
        grid=(N // 1000, D_OUT // 128),
        in_specs=[
            pl.BlockSpec((4, 1000, 128), lambda r, c: (0, r, 0)),
            pl.BlockSpec((4, 1000, 128), lambda r, c: (0, r, 0)),
            pl.BlockSpec((1000, 1), lambda r, c: (r, 0)),
            pl.BlockSpec((4, 128), lambda r, c: (0, 0)),
            pl.BlockSpec((D_HID, 128), lambda r, c: (0, c)),
        ],
        out_specs=pl.BlockSpec((1, 1000, 128), lambda r, c: (c, r, 0)),
        out_shape=jax.ShapeDtypeStruct((D_OUT // 128, N, 128), _f32),
    )(agg1, g1, dis_col, b1c, W2)


def _tc3_body(agg_ref, g_ref, dis_ref, b2_ref, out_ref):
    r = pl.program_id(0)
    dis = dis_ref[...]
    y = jnp.concatenate(
        [dis * (agg_ref[k] + g_ref[k]) + b2_ref[k] for k in range(2)], axis=1)
    p = jnp.sum(y, axis=0, keepdims=True) * (1.0 / (N // 2))

    @pl.when(r == 0)
    def _():
        out_ref[...] = jnp.zeros((2, D_OUT), _f32)

    gid = r // 5
    mask = lax.broadcasted_iota(_i32, (2, D_OUT), 0) == gid
    out_ref[...] += jnp.where(mask, jnp.broadcast_to(p, (2, D_OUT)), 0.0)


def _tc3(agg2, g2, dis_col, b2c):
    return pl.pallas_call(
        _tc3_body,
        grid=(N // 1000,),
        in_specs=[
            pl.BlockSpec((2, 1000, 128), lambda r: (0, r, 0)),
            pl.BlockSpec((2, 1000, 128), lambda r: (0, r, 0)),
            pl.BlockSpec((1000, 1), lambda r: (r, 0)),
            pl.BlockSpec((2, 128), lambda r: (0, 0)),
        ],
        out_specs=pl.BlockSpec((2, D_OUT), lambda r: (0, 0)),
        out_shape=jax.ShapeDtypeStruct((2, D_OUT), _f32),
    )(agg2, g2, dis_col, b2c)


def kernel(x, edge_index, W1, b1, W2, b2):
    row = edge_index[0]
    col = edge_index[1]

    dis = _make_deg_kernel()(col)
    dis_col = dis.reshape(N, 1)

    g1 = _tc1(x, W1, dis_col)
    agg1 = _make_scatter_kernel(4)(g1.reshape(4 * N, 128), row, col)
    g2 = _tc2(agg1.reshape(4, N, 128), g1, dis_col,
              b1.reshape(4, 128), W2)
    agg2 = _make_scatter_kernel(2)(g2.reshape(2 * N, 128), row, col)
    return _tc3(agg2.reshape(2, N, 128), g2, dis_col, b2.reshape(2, 128))


# speedup vs baseline: 15.8729x; 1.2739x over previous
import functools

import jax
import jax.numpy as jnp
from jax import lax
from jax.experimental import pallas as pl
from jax.experimental.pallas import tpu as pltpu
from jax.experimental.pallas import tpu_sc as plsc

N = 10000
E = 160000
D_IN = 256
D_HID = 512
D_OUT = 256

NC = 2
NS = 16
EPT = E // NS
NB = EPT // 128
TAIL = EPT - NB * 128
SLAB_STEP = 624
SLAB = 640
NBUF = 2

_f32 = jnp.float32
_i32 = jnp.int32


def _rsqrt16(d):
    i = lax.bitcast_convert_type(d, _i32)
    i = 0x5F3759DF - lax.shift_right_logical(i, 1)
    y = lax.bitcast_convert_type(i, _f32)
    for _ in range(3):
        y = y * (1.5 - 0.5 * d * y * y)
    return y


@functools.lru_cache(maxsize=None)
def _make_deg_kernel():
    mesh = plsc.VectorSubcoreMesh(core_axis_name="c", subcore_axis_name="s")

    @functools.partial(
        pl.kernel,
        out_type=jax.ShapeDtypeStruct((N,), _f32),
        mesh=mesh,
        scratch_types=[
            pltpu.VMEM((128,), _i32),
            pltpu.VMEM((16,), _i32),
            pltpu.VMEM((128,), _f32),
            pltpu.VMEM((SLAB,), _f32),
            pltpu.VMEM_SHARED((N,), _f32),
        ],
    )
    def deg_kernel(col_hbm, dis_hbm, idx_v, idx_t, ones_v, buf_v, acc):
        c = lax.axis_index("c")
        s = lax.axis_index("s")
        slab0 = s * SLAB_STEP

        @pl.when(c == 0)
        def _zero():
            for j in range(SLAB // 16):
                buf_v[pl.ds(j * 16, 16)] = jnp.zeros((16,), _f32)
            pltpu.sync_copy(buf_v, acc.at[pl.ds(slab0, SLAB)])

        plsc.subcore_barrier()

        @pl.when(c == 0)
        def _scatter():
            for k in range(8):
                ones_v[pl.ds(k * 16, 16)] = jnp.full((16,), 1.0, _f32)
            base = s * EPT

            def step(i, carry):
                off = base + i * 128
                pltpu.sync_copy(col_hbm.at[pl.ds(off, 128)], idx_v)
                pltpu.sync_copy(ones_v, acc.at[idx_v], add=True)
                return carry

            lax.fori_loop(0, NB, step, 0)
            pltpu.sync_copy(col_hbm.at[pl.ds(base + NB * 128, TAIL)], idx_t)
            pltpu.sync_copy(ones_v.at[pl.ds(0, TAIL)], acc.at[idx_t], add=True)

        plsc.subcore_barrier()

        @pl.when(c == 0)
        def _dis():
            pltpu.sync_copy(acc.at[pl.ds(slab0, SLAB)], buf_v)
            for j in range(SLAB // 16):
                d = buf_v[pl.ds(j * 16, 16)] + 1.0
                buf_v[pl.ds(j * 16, 16)] = _rsqrt16(d)
            pltpu.sync_copy(buf_v, dis_hbm.at[pl.ds(slab0, SLAB)])

    return deg_kernel


@functools.lru_cache(maxsize=None)
def _make_scatter_kernel(nchunk):
    cpc = nchunk // NC
    mesh = plsc.VectorSubcoreMesh(core_axis_name="c", subcore_axis_name="s")

    @functools.partial(
        pl.kernel,
        out_type=jax.ShapeDtypeStruct((nchunk * N, 128), _f32),
        mesh=mesh,
        scratch_types=[
            pltpu.VMEM((EPT,), _i32),
            pltpu.VMEM((NBUF, 128), _i32),
            pltpu.VMEM((16,), _i32),
            pltpu.VMEM((16,), _i32),
            pltpu.VMEM((NBUF, 128, 128), _f32),
            pltpu.VMEM_SHARED((N, 128), _f32),
            pltpu.SemaphoreType.DMA((NBUF,)),
        ],
    )
    def scatter_kernel(g_hbm, row_hbm, col_hbm, agg_hbm,
                       idx_s, idx_d, idx_st, idx_dt, bufs, acc, sems):
        c = lax.axis_index("c")
        s = lax.axis_index("s")
        slab0 = s * SLAB_STEP
        base = s * EPT

        pltpu.sync_copy(row_hbm.at[pl.ds(base, EPT)], idx_s)
        pltpu.sync_copy(row_hbm.at[pl.ds(base + NB * 128, TAIL)], idx_st)
        pltpu.sync_copy(col_hbm.at[pl.ds(base + NB * 128, TAIL)], idx_dt)

        def zfill(r, carry):
            for k in range(8):
                bufs[0, r, pl.ds(k * 16, 16)] = jnp.zeros((16,), _f32)
            return carry

        for j in range(cpc):
            chunk = c * cpc + j
            goff = chunk * N
            gview = g_hbm.at[pl.ds(goff, N)]

            lax.fori_loop(0, 128, zfill, 0)
            for p in range(SLAB // 128):
                pltpu.sync_copy(bufs.at[0], acc.at[pl.ds(slab0 + p * 128, 128)])
            plsc.subcore_barrier()

            def issue(off, b):
                pltpu.async_copy(col_hbm.at[pl.ds(base + off * 128, 128)],
                                 idx_d.at[b], sems.at[b])
                pltpu.async_copy(gview.at[idx_s.at[pl.ds(off * 128, 128)]],
                                 bufs.at[b], sems.at[b])

            def drain(b):
                pltpu.make_async_copy(col_hbm.at[pl.ds(0, 128)],
                                      idx_d.at[b], sems.at[b]).wait()
                pltpu.make_async_copy(g_hbm.at[pl.ds(0, 128)],
                                      bufs.at[b], sems.at[b]).wait()

            for b in range(NBUF):
                issue(b, b)

            def body(t, carry):
                i0 = t * NBUF
                for b in range(NBUF):
                    drain(b)
                    pltpu.sync_copy(bufs.at[b], acc.at[idx_d.at[b]], add=True)
                    nxt = i0 + NBUF + b

                    @pl.when(nxt < NB)
                    def _():
                        issue(nxt, b)
                return carry

            lax.fori_loop(0, NB // NBUF, body, 0)

            pltpu.sync_copy(gview.at[idx_st], bufs.at[0].at[pl.ds(0, TAIL)])
            pltpu.sync_copy(bufs.at[0].at[pl.ds(0, TAIL)],
                            acc.at[idx_dt], add=True)

            plsc.subcore_barrier()
            for p in range(SLAB // 128):
                pltpu.sync_copy(acc.at[pl.ds(slab0 + p * 128, 128)], bufs.at[0])
                pltpu.sync_copy(bufs.at[0],
                                agg_hbm.at[pl.ds(goff + slab0 + p * 128, 128)])
            plsc.subcore_barrier()

    return scatter_kernel


def _tc1_body(x_ref, w_ref, dis_ref, out_ref):
    h = jnp.dot(x_ref[...], w_ref[...], preferred_element_type=_f32)
    out_ref[0] = h * dis_ref[...]


def _tc1(x, W1, dis_col):
    return pl.pallas_call(
        _tc1_body,
        grid=(N // 1000, D_HID // 128),
        in_specs=[
            pl.BlockSpec((1000, D_IN), lambda r, c: (r, 0)),
            pl.BlockSpec((D_IN, 128), lambda r, c: (0, c)),
            pl.BlockSpec((1000, 1), lambda r, c: (r, 0)),
        ],
        out_specs=pl.BlockSpec((1, 1000, 128), lambda r, c: (c, r, 0)),
        out_shape=jax.ShapeDtypeStruct((D_HID // 128, N, 128), _f32),
    )(x, W1, dis_col)


def _tc2_body(agg_ref, g_ref, dis_ref, b1_ref, w2_ref, out_ref):
    dis = dis_ref[...]
    z = jnp.concatenate(
        [jax.nn.relu(dis * (agg_ref[k] + g_ref[k]) + b1_ref[k]) for k in range(4)],
        axis=1)
    h2 = jnp.dot(z, w2_ref[...], preferred_element_type=_f32)
    out_ref[0] = h2 * dis


def _tc2(agg1, g1, dis_col, b1c, W2):
    return pl.pallas_call(
        _tc2_body,
        grid=(N // 1000, D_OUT // 128),
        in_specs=[
            pl.BlockSpec((4, 1000, 128), lambda r, c: (0, r, 0)),
            pl.BlockSpec((4, 1000, 128), lambda r, c: (0, r, 0)),
            pl.BlockSpec((1000, 1), lambda r, c: (r, 0)),
            pl.BlockSpec((4, 128), lambda r, c: (0, 0)),
            pl.BlockSpec((D_HID, 128), lambda r, c: (0, c)),
        ],
        out_specs=pl.BlockSpec((1, 1000, 128), lambda r, c: (c, r, 0)),
        out_shape=jax.ShapeDtypeStruct((D_OUT // 128, N, 128), _f32),
    )(agg1, g1, dis_col, b1c, W2)


def _tc3_body(agg_ref, g_ref, dis_ref, b2_ref, out_ref):
    r = pl.program_id(0)
    dis = dis_ref[...]
    y = jnp.concatenate(
        [dis * (agg_ref[k] + g_ref[k]) + b2_ref[k] for k in range(2)], axis=1)
    p = jnp.sum(y, axis=0, keepdims=True) * (1.0 / (N // 2))

    @pl.when(r == 0)
    def _():
        out_ref[...] = jnp.zeros((2, D_OUT), _f32)

    gid = r // 5
    mask = lax.broadcasted_iota(_i32, (2, D_OUT), 0) == gid
    out_ref[...] += jnp.where(mask, jnp.broadcast_to(p, (2, D_OUT)), 0.0)


def _tc3(agg2, g2, dis_col, b2c):
    return pl.pallas_call(
        _tc3_body,
        grid=(N // 1000,),
        in_specs=[
            pl.BlockSpec((2, 1000, 128), lambda r: (0, r, 0)),
            pl.BlockSpec((2, 1000, 128), lambda r: (0, r, 0)),
            pl.BlockSpec((1000, 1), lambda r: (r, 0)),
            pl.BlockSpec((2, 128), lambda r: (0, 0)),
        ],
        out_specs=pl.BlockSpec((2, D_OUT), lambda r: (0, 0)),
        out_shape=jax.ShapeDtypeStruct((2, D_OUT), _f32),
    )(agg2, g2, dis_col, b2c)


def kernel(x, edge_index, W1, b1, W2, b2):
    row = edge_index[0]
    col = edge_index[1]

    dis = _make_deg_kernel()(col)
    dis_col = dis.reshape(N, 1)

    g1 = _tc1(x, W1, dis_col)
    agg1 = _make_scatter_kernel(4)(g1.reshape(4 * N, 128), row, col)
    g2 = _tc2(agg1.reshape(4, N, 128), g1, dis_col,
              b1.reshape(4, 128), W2)
    agg2 = _make_scatter_kernel(2)(g2.reshape(2 * N, 128), row, col)
    return _tc3(agg2.reshape(2, N, 128), g2, dis_col, b2.reshape(2, 128))


# speedup vs baseline: 16.5929x; 1.0454x over previous
import functools

import jax
import jax.numpy as jnp
from jax import lax
from jax.experimental import pallas as pl
from jax.experimental.pallas import tpu as pltpu
from jax.experimental.pallas import tpu_sc as plsc

N = 10000
E = 160000
D_IN = 256
D_HID = 512
D_OUT = 256

NC = 2
NS = 16
EPT = E // NS
NB = EPT // 128
TAIL = EPT - NB * 128
SLAB_STEP = 624
SLAB = 640
NBUF = 2
NBUF_D = 6

_f32 = jnp.float32
_i32 = jnp.int32


def _rsqrt16(d):
    i = lax.bitcast_convert_type(d, _i32)
    i = 0x5F3759DF - lax.shift_right_logical(i, 1)
    y = lax.bitcast_convert_type(i, _f32)
    for _ in range(3):
        y = y * (1.5 - 0.5 * d * y * y)
    return y


@functools.lru_cache(maxsize=None)
def _make_deg_kernel():
    mesh = plsc.VectorSubcoreMesh(core_axis_name="c", subcore_axis_name="s")

    @functools.partial(
        pl.kernel,
        out_type=jax.ShapeDtypeStruct((N,), _f32),
        mesh=mesh,
        scratch_types=[
            pltpu.VMEM((NBUF_D, 128), _i32),
            pltpu.VMEM((16,), _i32),
            pltpu.VMEM((128,), _f32),
            pltpu.VMEM((SLAB,), _f32),
            pltpu.VMEM_SHARED((N,), _f32),
            pltpu.SemaphoreType.DMA((NBUF_D,)),
        ],
    )
    def deg_kernel(col_hbm, dis_hbm, idx_v, idx_t, ones_v, buf_v, acc, sems):
        c = lax.axis_index("c")
        s = lax.axis_index("s")
        slab0 = s * SLAB_STEP

        @pl.when(c == 0)
        def _zero():
            for j in range(SLAB // 16):
                buf_v[pl.ds(j * 16, 16)] = jnp.zeros((16,), _f32)
            pltpu.sync_copy(buf_v, acc.at[pl.ds(slab0, SLAB)])

        plsc.subcore_barrier()

        @pl.when(c == 0)
        def _scatter():
            for k in range(8):
                ones_v[pl.ds(k * 16, 16)] = jnp.full((16,), 1.0, _f32)
            base = s * EPT

            for b in range(NBUF_D):
                pltpu.async_copy(col_hbm.at[pl.ds(base + b * 128, 128)],
                                 idx_v.at[b], sems.at[b])

            def step(t, carry):
                i0 = t * NBUF_D
                for b in range(NBUF_D):
                    pltpu.make_async_copy(col_hbm.at[pl.ds(0, 128)],
                                          idx_v.at[b], sems.at[b]).wait()
                    pltpu.sync_copy(ones_v, acc.at[idx_v.at[b]], add=True)
                    nxt = i0 + NBUF_D + b

                    @pl.when(nxt < NB)
                    def _():
                        pltpu.async_copy(
                            col_hbm.at[pl.ds(base + nxt * 128, 128)],
                            idx_v.at[b], sems.at[b])
                return carry

            lax.fori_loop(0, NB // NBUF_D, step, 0)
            pltpu.sync_copy(col_hbm.at[pl.ds(base + NB * 128, TAIL)], idx_t)
            pltpu.sync_copy(ones_v.at[pl.ds(0, TAIL)], acc.at[idx_t], add=True)

        plsc.subcore_barrier()

        @pl.when(c == 0)
        def _dis():
            pltpu.sync_copy(acc.at[pl.ds(slab0, SLAB)], buf_v)
            for j in range(SLAB // 16):
                d = buf_v[pl.ds(j * 16, 16)] + 1.0
                buf_v[pl.ds(j * 16, 16)] = _rsqrt16(d)
            pltpu.sync_copy(buf_v, dis_hbm.at[pl.ds(slab0, SLAB)])

    return deg_kernel


@functools.lru_cache(maxsize=None)
def _make_scatter_kernel(nchunk):
    cpc = nchunk // NC
    mesh = plsc.VectorSubcoreMesh(core_axis_name="c", subcore_axis_name="s")

    @functools.partial(
        pl.kernel,
        out_type=jax.ShapeDtypeStruct((nchunk * N, 128), _f32),
        mesh=mesh,
        scratch_types=[
            pltpu.VMEM((EPT,), _i32),
            pltpu.VMEM((NBUF, 128), _i32),
            pltpu.VMEM((16,), _i32),
            pltpu.VMEM((16,), _i32),
            pltpu.VMEM((NBUF, 128, 128), _f32),
            pltpu.VMEM_SHARED((N, 128), _f32),
            pltpu.SemaphoreType.DMA((NBUF,)),
        ],
    )
    def scatter_kernel(g_hbm, row_hbm, col_hbm, agg_hbm,
                       idx_s, idx_d, idx_st, idx_dt, bufs, acc, sems):
        c = lax.axis_index("c")
        s = lax.axis_index("s")
        slab0 = s * SLAB_STEP
        base = s * EPT

        pltpu.sync_copy(row_hbm.at[pl.ds(base, EPT)], idx_s)
        pltpu.sync_copy(row_hbm.at[pl.ds(base + NB * 128, TAIL)], idx_st)
        pltpu.sync_copy(col_hbm.at[pl.ds(base + NB * 128, TAIL)], idx_dt)

        def zfill(r, carry):
            for k in range(8):
                bufs[0, r, pl.ds(k * 16, 16)] = jnp.zeros((16,), _f32)
            return carry

        for j in range(cpc):
            chunk = c * cpc + j
            goff = chunk * N
            gview = g_hbm.at[pl.ds(goff, N)]

            lax.fori_loop(0, 128, zfill, 0)
            for p in range(SLAB // 128):
                pltpu.async_copy(bufs.at[0], acc.at[pl.ds(slab0 + p * 128, 128)],
                                 sems.at[p % NBUF])
            for p in range(SLAB // 128):
                pltpu.make_async_copy(bufs.at[0],
                                      acc.at[pl.ds(slab0, 128)],
                                      sems.at[p % NBUF]).wait()
            plsc.subcore_barrier()

            def issue(off, b):
                pltpu.async_copy(col_hbm.at[pl.ds(base + off * 128, 128)],
                                 idx_d.at[b], sems.at[b])
                pltpu.async_copy(gview.at[idx_s.at[pl.ds(off * 128, 128)]],
                                 bufs.at[b], sems.at[b])

            def drain(b):
                pltpu.make_async_copy(col_hbm.at[pl.ds(0, 128)],
                                      idx_d.at[b], sems.at[b]).wait()
                pltpu.make_async_copy(g_hbm.at[pl.ds(0, 128)],
                                      bufs.at[b], sems.at[b]).wait()

            for b in range(NBUF):
                issue(b, b)

            def body(t, carry):
                i0 = t * NBUF
                for b in range(NBUF):
                    drain(b)
                    pltpu.sync_copy(bufs.at[b], acc.at[idx_d.at[b]], add=True)
                    nxt = i0 + NBUF + b

                    @pl.when(nxt < NB)
                    def _():
                        issue(nxt, b)
                return carry

            lax.fori_loop(0, NB // NBUF, body, 0)

            pltpu.sync_copy(gview.at[idx_st], bufs.at[0].at[pl.ds(0, TAIL)])
            pltpu.sync_copy(bufs.at[0].at[pl.ds(0, TAIL)],
                            acc.at[idx_dt], add=True)

            plsc.subcore_barrier()
            for p in range(SLAB // 128):
                b = p % NBUF
                if p >= NBUF:
                    pltpu.make_async_copy(bufs.at[b],
                                          agg_hbm.at[pl.ds(0, 128)],
                                          sems.at[b]).wait()
                pltpu.sync_copy(acc.at[pl.ds(slab0 + p * 128, 128)], bufs.at[b])
                pltpu.async_copy(bufs.at[b],
                                 agg_hbm.at[pl.ds(goff + slab0 + p * 128, 128)],
                                 sems.at[b])
            for b in range(NBUF):
                pltpu.make_async_copy(bufs.at[b], agg_hbm.at[pl.ds(0, 128)],
                                      sems.at[b]).wait()
            plsc.subcore_barrier()

    return scatter_kernel


def _tc1m_body(x_ref, w_ref, out_ref):
    out_ref[0] = jnp.dot(x_ref[...], w_ref[...], preferred_element_type=_f32)


def _tc1m(x, W1):
    return pl.pallas_call(
        _tc1m_body,
        grid=(N // 1000, D_HID // 128),
        in_specs=[
            pl.BlockSpec((1000, D_IN), lambda r, c: (r, 0)),
            pl.BlockSpec((D_IN, 128), lambda r, c: (0, c)),
        ],
        out_specs=pl.BlockSpec((1, 1000, 128), lambda r, c: (c, r, 0)),
        out_shape=jax.ShapeDtypeStruct((D_HID // 128, N, 128), _f32),
    )(x, W1)


def _tc1s_body(h_ref, dis_ref, out_ref):
    out_ref[...] = h_ref[...] * dis_ref[...]


def _tc1s(h1, dis_col):
    return pl.pallas_call(
        _tc1s_body,
        grid=(N // 1000, D_HID // 128),
        in_specs=[
            pl.BlockSpec((1, 1000, 128), lambda r, c: (c, r, 0)),
            pl.BlockSpec((1000, 1), lambda r, c: (r, 0)),
        ],
        out_specs=pl.BlockSpec((1, 1000, 128), lambda r, c: (c, r, 0)),
        out_shape=jax.ShapeDtypeStruct((D_HID // 128, N, 128), _f32),
    )(h1, dis_col)


def _tc2_body(agg_ref, g_ref, dis_ref, b1_ref, w2_ref, out_ref):
    dis = dis_ref[...]
    z = jnp.concatenate(
        [jax.nn.relu(dis * (agg_ref[k] + g_ref[k]) + b1_ref[k]) for k in range(4)],
        axis=1)
    h2 = jnp.dot(z, w2_ref[...], preferred_element_type=_f32)
    out_ref[0] = h2 * dis


def _tc2(agg1, g1, dis_col, b1c, W2):
    return pl.pallas_call(
        _tc2_body,
        grid=(N // 1000, D_OUT // 128),
        in_specs=[
            pl.BlockSpec((4, 1000, 128), lambda r, c: (0, r, 0)),
            pl.BlockSpec((4, 1000, 128), lambda r, c: (0, r, 0)),
            pl.BlockSpec((1000, 1), lambda r, c: (r, 0)),
            pl.BlockSpec((4, 128), lambda r, c: (0, 0)),
            pl.BlockSpec((D_HID, 128), lambda r, c: (0, c)),
        ],
        out_specs=pl.BlockSpec((1, 1000, 128), lambda r, c: (c, r, 0)),
        out_shape=jax.ShapeDtypeStruct((D_OUT // 128, N, 128), _f32),
    )(agg1, g1, dis_col, b1c, W2)


def _tc3_body(agg_ref, g_ref, dis_ref, b2_ref, out_ref):
    r = pl.program_id(0)
    dis = dis_ref[...]
    y = jnp.concatenate(
        [dis * (agg_ref[k] + g_ref[k]) + b2_ref[k] for k in range(2)], axis=1)
    p = jnp.sum(y, axis=0, keepdims=True) * (1.0 / (N // 2))

    @pl.when(r == 0)
    def _():
        out_ref[...] = jnp.zeros((2, D_OUT), _f32)

    gid = r // 5
    mask = lax.broadcasted_iota(_i32, (2, D_OUT), 0) == gid
    out_ref[...] += jnp.where(mask, jnp.broadcast_to(p, (2, D_OUT)), 0.0)


def _tc3(agg2, g2, dis_col, b2c):
    return pl.pallas_call(
        _tc3_body,
        grid=(N // 1000,),
        in_specs=[
            pl.BlockSpec((2, 1000, 128), lambda r: (0, r, 0)),
            pl.BlockSpec((2, 1000, 128), lambda r: (0, r, 0)),
            pl.BlockSpec((1000, 1), lambda r: (r, 0)),
            pl.BlockSpec((2, 128), lambda r: (0, 0)),
        ],
        out_specs=pl.BlockSpec((2, D_OUT), lambda r: (0, 0)),
        out_shape=jax.ShapeDtypeStruct((2, D_OUT), _f32),
    )(agg2, g2, dis_col, b2c)


def kernel(x, edge_index, W1, b1, W2, b2):
    row = edge_index[0]
    col = edge_index[1]

    dis = _make_deg_kernel()(col)
    h1 = _tc1m(x, W1)
    dis_col = dis.reshape(N, 1)

    g1 = _tc1s(h1, dis_col)
    agg1 = _make_scatter_kernel(4)(g1.reshape(4 * N, 128), row, col)
    g2 = _tc2(agg1.reshape(4, N, 128), g1, dis_col,
              b1.reshape(4, 128), W2)
    agg2 = _make_scatter_kernel(2)(g2.reshape(2 * N, 128), row, col)
    return _tc3(agg2.reshape(2, N, 128), g2, dis_col, b2.reshape(2, 128))


# speedup vs baseline: 17.3608x; 1.0463x over previous
import functools

import jax
import jax.numpy as jnp
from jax import lax
from jax.experimental import pallas as pl
from jax.experimental.pallas import tpu as pltpu
from jax.experimental.pallas import tpu_sc as plsc

N = 10000
E = 160000
D_IN = 256
D_HID = 512
D_OUT = 256

NC = 2
NS = 16
EPT = E // NS
NB = EPT // 128
TAIL = EPT - NB * 128
SLAB_STEP = 624
SLAB = 640
NBUF = 2
NBUF_D = 6

_f32 = jnp.float32
_i32 = jnp.int32


def _rsqrt16(d):
    i = lax.bitcast_convert_type(d, _i32)
    i = 0x5F3759DF - lax.shift_right_logical(i, 1)
    y = lax.bitcast_convert_type(i, _f32)
    for _ in range(3):
        y = y * (1.5 - 0.5 * d * y * y)
    return y


@functools.lru_cache(maxsize=None)
def _make_deg_kernel():
    mesh = plsc.VectorSubcoreMesh(core_axis_name="c", subcore_axis_name="s")

    @functools.partial(
        pl.kernel,
        out_type=jax.ShapeDtypeStruct((N,), _f32),
        mesh=mesh,
        scratch_types=[
            pltpu.VMEM((NBUF_D, 128), _i32),
            pltpu.VMEM((16,), _i32),
            pltpu.VMEM((128,), _f32),
            pltpu.VMEM((SLAB,), _f32),
            pltpu.VMEM_SHARED((N,), _f32),
            pltpu.SemaphoreType.DMA((NBUF_D,)),
        ],
    )
    def deg_kernel(col_hbm, dis_hbm, idx_v, idx_t, ones_v, buf_v, acc, sems):
        c = lax.axis_index("c")
        s = lax.axis_index("s")
        slab0 = s * SLAB_STEP

        @pl.when(c == 0)
        def _zero():
            for j in range(SLAB // 16):
                buf_v[pl.ds(j * 16, 16)] = jnp.zeros((16,), _f32)
            pltpu.sync_copy(buf_v, acc.at[pl.ds(slab0, SLAB)])

        plsc.subcore_barrier()

        @pl.when(c == 0)
        def _scatter():
            for k in range(8):
                ones_v[pl.ds(k * 16, 16)] = jnp.full((16,), 1.0, _f32)
            base = s * EPT

            for b in range(NBUF_D):
                pltpu.async_copy(col_hbm.at[pl.ds(base + b * 128, 128)],
                                 idx_v.at[b], sems.at[b])

            def step(t, carry):
                i0 = t * NBUF_D
                for b in range(NBUF_D):
                    pltpu.make_async_copy(col_hbm.at[pl.ds(0, 128)],
                                          idx_v.at[b], sems.at[b]).wait()
                    pltpu.sync_copy(ones_v, acc.at[idx_v.at[b]], add=True)
                    nxt = i0 + NBUF_D + b

                    @pl.when(nxt < NB)
                    def _():
                        pltpu.async_copy(
                            col_hbm.at[pl.ds(base + nxt * 128, 128)],
                            idx_v.at[b], sems.at[b])
                return carry

            lax.fori_loop(0, NB // NBUF_D, step, 0)
            pltpu.sync_copy(col_hbm.at[pl.ds(base + NB * 128, TAIL)], idx_t)
            pltpu.sync_copy(ones_v.at[pl.ds(0, TAIL)], acc.at[idx_t], add=True)

        plsc.subcore_barrier()

        @pl.when(c == 0)
        def _dis():
            pltpu.sync_copy(acc.at[pl.ds(slab0, SLAB)], buf_v)
            for j in range(SLAB // 16):
                d = buf_v[pl.ds(j * 16, 16)] + 1.0
                buf_v[pl.ds(j * 16, 16)] = _rsqrt16(d)
            pltpu.sync_copy(buf_v, dis_hbm.at[pl.ds(slab0, SLAB)])

    return deg_kernel


@functools.lru_cache(maxsize=None)
def _make_scatter_kernel(nchunk):
    cpc = nchunk // NC
    mesh = plsc.VectorSubcoreMesh(core_axis_name="c", subcore_axis_name="s")

    @functools.partial(
        pl.kernel,
        out_type=jax.ShapeDtypeStruct((nchunk * N, 128), _f32),
        mesh=mesh,
        scratch_types=[
            pltpu.VMEM((EPT,), _i32),
            pltpu.VMEM((NBUF, 128), _i32),
            pltpu.VMEM((16,), _i32),
            pltpu.VMEM((16,), _i32),
            pltpu.VMEM((NBUF, 128, 128), _f32),
            pltpu.VMEM_SHARED((N, 128), _f32),
            pltpu.SemaphoreType.DMA((NBUF,)),
        ],
    )
    def scatter_kernel(g_hbm, row_hbm, col_hbm, agg_hbm,
                       idx_s, idx_d, idx_st, idx_dt, bufs, acc, sems):
        c = lax.axis_index("c")
        s = lax.axis_index("s")
        slab0 = s * SLAB_STEP
        base = s * EPT

        pltpu.sync_copy(row_hbm.at[pl.ds(base, EPT)], idx_s)
        pltpu.sync_copy(row_hbm.at[pl.ds(base + NB * 128, TAIL)], idx_st)
        pltpu.sync_copy(col_hbm.at[pl.ds(base + NB * 128, TAIL)], idx_dt)

        def zfill(r, carry):
            for k in range(8):
                bufs[0, r, pl.ds(k * 16, 16)] = jnp.zeros((16,), _f32)
            return carry

        for j in range(cpc):
            chunk = c * cpc + j
            goff = chunk * N
            gview = g_hbm.at[pl.ds(goff, N)]

            lax.fori_loop(0, 128, zfill, 0)
            for p in range(SLAB // 128):
                pltpu.async_copy(bufs.at[0], acc.at[pl.ds(slab0 + p * 128, 128)],
                                 sems.at[p % NBUF])
            for p in range(SLAB // 128):
                pltpu.make_async_copy(bufs.at[0],
                                      acc.at[pl.ds(slab0, 128)],
                                      sems.at[p % NBUF]).wait()
            plsc.subcore_barrier()

            def issue(off, b):
                pltpu.async_copy(col_hbm.at[pl.ds(base + off * 128, 128)],
                                 idx_d.at[b], sems.at[b])
                pltpu.async_copy(gview.at[idx_s.at[pl.ds(off * 128, 128)]],
                                 bufs.at[b], sems.at[b])

            def drain(b):
                pltpu.make_async_copy(col_hbm.at[pl.ds(0, 128)],
                                      idx_d.at[b], sems.at[b]).wait()
                pltpu.make_async_copy(g_hbm.at[pl.ds(0, 128)],
                                      bufs.at[b], sems.at[b]).wait()

            for b in range(NBUF):
                issue(b, b)

            def body(t, carry):
                i0 = t * NBUF
                for b in range(NBUF):
                    drain(b)
                    pltpu.sync_copy(bufs.at[b], acc.at[idx_d.at[b]], add=True)
                    nxt = i0 + NBUF + b

                    @pl.when(nxt < NB)
                    def _():
                        issue(nxt, b)
                return carry

            lax.fori_loop(0, NB // NBUF, body, 0)

            pltpu.sync_copy(gview.at[idx_st], bufs.at[0].at[pl.ds(0, TAIL)])
            pltpu.sync_copy(bufs.at[0].at[pl.ds(0, TAIL)],
                            acc.at[idx_dt], add=True)

            plsc.subcore_barrier()
            for p in range(SLAB // 128):
                b = p % NBUF
                if p >= NBUF:
                    pltpu.make_async_copy(bufs.at[b],
                                          agg_hbm.at[pl.ds(0, 128)],
                                          sems.at[b]).wait()
                pltpu.sync_copy(acc.at[pl.ds(slab0 + p * 128, 128)], bufs.at[b])
                pltpu.async_copy(bufs.at[b],
                                 agg_hbm.at[pl.ds(goff + slab0 + p * 128, 128)],
                                 sems.at[b])
            for b in range(NBUF):
                pltpu.make_async_copy(bufs.at[b], agg_hbm.at[pl.ds(0, 128)],
                                      sems.at[b]).wait()
            plsc.subcore_barrier()

    return scatter_kernel


def _tc1_body(x_ref, w_ref, dis_ref, out_ref):
    h = jnp.dot(x_ref[...], w_ref[...], preferred_element_type=_f32)
    out_ref[0] = h * dis_ref[...]


def _tc1(x, W1, dis_col):
    return pl.pallas_call(
        _tc1_body,
        grid=(N // 1000, D_HID // 128),
        in_specs=[
            pl.BlockSpec((1000, D_IN), lambda r, c: (r, 0)),
            pl.BlockSpec((D_IN, 128), lambda r, c: (0, c)),
            pl.BlockSpec((1000, 1), lambda r, c: (r, 0)),
        ],
        out_specs=pl.BlockSpec((1, 1000, 128), lambda r, c: (c, r, 0)),
        out_shape=jax.ShapeDtypeStruct((D_HID // 128, N, 128), _f32),
    )(x, W1, dis_col)


def _tc2_body(agg_ref, g_ref, dis_ref, b1_ref, w2_ref, out_ref):
    dis = dis_ref[...]
    z = jnp.concatenate(
        [jax.nn.relu(dis * (agg_ref[k] + g_ref[k]) + b1_ref[k]) for k in range(4)],
        axis=1)
    h2 = jnp.dot(z, w2_ref[...], preferred_element_type=_f32)
    out_ref[0] = h2 * dis


def _tc2(agg1, g1, dis_col, b1c, W2):
    return pl.pallas_call(
        _tc2_body,
        grid=(N // 1000, D_OUT // 128),
        in_specs=[
            pl.BlockSpec((4, 1000, 128), lambda r, c: (0, r, 0)),
            pl.BlockSpec((4, 1000, 128), lambda r, c: (0, r, 0)),
            pl.BlockSpec((1000, 1), lambda r, c: (r, 0)),
            pl.BlockSpec((4, 128), lambda r, c: (0, 0)),
            pl.BlockSpec((D_HID, 128), lambda r, c: (0, c)),
        ],
        out_specs=pl.BlockSpec((1, 1000, 128), lambda r, c: (c, r, 0)),
        out_shape=jax.ShapeDtypeStruct((D_OUT // 128, N, 128), _f32),
    )(agg1, g1, dis_col, b1c, W2)


def _tc3_body(agg_ref, g_ref, dis_ref, b2_ref, out_ref):
    r = pl.program_id(0)
    dis = dis_ref[...]
    y = jnp.concatenate(
        [dis * (agg_ref[k] + g_ref[k]) + b2_ref[k] for k in range(2)], axis=1)
    p = jnp.sum(y, axis=0, keepdims=True) * (1.0 / (N // 2))

    @pl.when(r == 0)
    def _():
        out_ref[...] = jnp.zeros((2, D_OUT), _f32)

    gid = r // 5
    mask = lax.broadcasted_iota(_i32, (2, D_OUT), 0) == gid
    out_ref[...] += jnp.where(mask, jnp.broadcast_to(p, (2, D_OUT)), 0.0)


def _tc3(agg2, g2, dis_col, b2c):
    return pl.pallas_call(
        _tc3_body,
        grid=(N // 1000,),
        in_specs=[
            pl.BlockSpec((2, 1000, 128), lambda r: (0, r, 0)),
            pl.BlockSpec((2, 1000, 128), lambda r: (0, r, 0)),
            pl.BlockSpec((1000, 1), lambda r: (r, 0)),
            pl.BlockSpec((2, 128), lambda r: (0, 0)),
        ],
        out_specs=pl.BlockSpec((2, D_OUT), lambda r: (0, 0)),
        out_shape=jax.ShapeDtypeStruct((2, D_OUT), _f32),
    )(agg2, g2, dis_col, b2c)


def kernel(x, edge_index, W1, b1, W2, b2):
    row = edge_index[0]
    col = edge_index[1]

    dis = _make_deg_kernel()(col)
    dis_col = dis.reshape(N, 1)

    g1 = _tc1(x, W1, dis_col)
    agg1 = _make_scatter_kernel(4)(g1.reshape(4 * N, 128), row, col)
    g2 = _tc2(agg1.reshape(4, N, 128), g1, dis_col,
              b1.reshape(4, 128), W2)
    agg2 = _make_scatter_kernel(2)(g2.reshape(2 * N, 128), row, col)
    return _tc3(agg2.reshape(2, N, 128), g2, dis_col, b2.reshape(2, 128))
